# speedup vs baseline: 1.2572x; 1.2572x over previous
import functools

import jax
import jax.numpy as jnp
from jax import lax
from jax.experimental import pallas as pl
from jax.experimental.pallas import tpu as pltpu
from jax.experimental.pallas import tpu_sc as plsc

NUM_CORES = 2
NUM_SUBCORES = 16
NUM_WORKERS = NUM_CORES * NUM_SUBCORES
CHUNK = 64


@functools.partial(jax.jit, static_argnames=("n_per_w", "n_chunks", "dim"))
def _gather_call(idx_flat, table, *, n_per_w, n_chunks, dim):
    n_total = idx_flat.shape[0]
    mesh = plsc.VectorSubcoreMesh(core_axis_name="c", subcore_axis_name="s")

    @functools.partial(
        pl.kernel,
        out_type=jax.ShapeDtypeStruct((n_total, dim), jnp.float32),
        mesh=mesh,
        scratch_types=[
            pltpu.VMEM((n_per_w,), jnp.int32),
            pltpu.VMEM((CHUNK, dim), jnp.float32),
            pltpu.SemaphoreType.DMA,
        ],
    )
    def gather_kernel(idx_hbm, table_hbm, out_hbm, idx_v, rows_v, sem):
        wid = lax.axis_index("s") * NUM_CORES + lax.axis_index("c")
        base = wid * n_per_w
        pltpu.sync_copy(idx_hbm.at[pl.ds(base, n_per_w)], idx_v)

        def body(c, carry):
            start = c * CHUNK
            pltpu.async_copy(
                table_hbm.at[idx_v.at[pl.ds(start, CHUNK)]],
                rows_v,
                sem,
            ).wait()
            pltpu.sync_copy(rows_v, out_hbm.at[pl.ds(base + start, CHUNK)])
            return carry

        lax.fori_loop(0, n_chunks, body, 0)

    return gather_kernel(idx_flat, table)


def kernel(x, table):
    b, l = x.shape
    _, dim = table.shape
    n_total = b * l
    n_per_w = n_total // NUM_WORKERS
    n_chunks = n_per_w // CHUNK
    idx_flat = x.reshape(n_total)
    out = _gather_call(idx_flat, table, n_per_w=n_per_w, n_chunks=n_chunks, dim=dim)
    return out.reshape(b, l, dim)


# speedup vs baseline: 1.3066x; 1.0393x over previous
import functools

import jax
import jax.numpy as jnp
from jax import lax
from jax.experimental import pallas as pl
from jax.experimental.pallas import tpu as pltpu
from jax.experimental.pallas import tpu_sc as plsc

NUM_CORES = 2
NUM_SUBCORES = 16
NUM_WORKERS = NUM_CORES * NUM_SUBCORES
CHUNK = 64


@functools.partial(jax.jit, static_argnames=("n_per_w", "n_chunks", "dim"))
def _gather_call(idx_flat, table, *, n_per_w, n_chunks, dim):
    n_total = idx_flat.shape[0]
    mesh = plsc.VectorSubcoreMesh(core_axis_name="c", subcore_axis_name="s")

    @functools.partial(
        pl.kernel,
        out_type=jax.ShapeDtypeStruct((n_total, dim), jnp.float32),
        mesh=mesh,
        scratch_types=[
            pltpu.VMEM((n_per_w,), jnp.int32),
            pltpu.VMEM((2, CHUNK, dim), jnp.float32),
            pltpu.SemaphoreType.DMA,
            pltpu.SemaphoreType.DMA,
        ],
    )
    def gather_kernel(idx_hbm, table_hbm, out_hbm, idx_v, rows_v, sem0, sem1):
        wid = lax.axis_index("s") * NUM_CORES + lax.axis_index("c")
        base = wid * n_per_w
        pltpu.sync_copy(idx_hbm.at[pl.ds(base, n_per_w)], idx_v)

        bufs = (rows_v.at[0], rows_v.at[1])
        sems = (sem0, sem1)

        def start_gather(c, b):
            pltpu.async_copy(
                table_hbm.at[idx_v.at[pl.ds(c * CHUNK, CHUNK)]],
                bufs[b],
                sems[b],
            )

        def wait_gather(b):
            pltpu.make_async_copy(
                table_hbm.at[pl.ds(0, CHUNK)], bufs[b], sems[b]
            ).wait()

        start_gather(0, 0)

        def body(i, carry):
            c = i * 2
            start_gather(c + 1, 1)
            wait_gather(0)
            pltpu.sync_copy(bufs[0], out_hbm.at[pl.ds(base + c * CHUNK, CHUNK)])

            @pl.when(c + 2 < n_chunks)
            def _():
                start_gather(c + 2, 0)

            wait_gather(1)
            pltpu.sync_copy(
                bufs[1], out_hbm.at[pl.ds(base + (c + 1) * CHUNK, CHUNK)]
            )
            return carry

        lax.fori_loop(0, n_chunks // 2, body, 0)

    return gather_kernel(idx_flat, table)


def kernel(x, table):
    b, l = x.shape
    _, dim = table.shape
    n_total = b * l
    n_per_w = n_total // NUM_WORKERS
    n_chunks = n_per_w // CHUNK
    idx_flat = x.reshape(n_total)
    out = _gather_call(idx_flat, table, n_per_w=n_per_w, n_chunks=n_chunks, dim=dim)
    return out.reshape(b, l, dim)
